# baseline (speedup 1.0000x reference)
import functools

import jax
import jax.numpy as jnp
from jax import lax
from jax.experimental import pallas as pl
from jax.experimental.pallas import tpu as pltpu
from jax.experimental.pallas import tpu_sc as plsc

_EMB = 64
_SCALE = _EMB ** (-0.5)
_NW = 32
_CHUNK = 128
_LANES = 16


def _sc_embed(x2d, table, n_rows_per_w, n_chunks_per_w):
    b_per_w = n_rows_per_w
    mesh = plsc.VectorSubcoreMesh(core_axis_name="c", subcore_axis_name="s")
    total = _NW * b_per_w

    @functools.partial(
        pl.kernel,
        mesh=mesh,
        compiler_params=pltpu.CompilerParams(use_tc_tiling_on_sc=False),
        out_type=jax.ShapeDtypeStruct((total, _EMB), jnp.float32),
        scratch_types=[
            pltpu.VMEM((n_chunks_per_w, _CHUNK), jnp.int32),
            pltpu.VMEM((_CHUNK, _EMB), jnp.float32),
            pltpu.SemaphoreType.DMA,
        ],
    )
    def k(x_hbm, table_hbm, out_hbm, idx_v, rows_v, sem):
        wid = lax.axis_index("s") * 2 + lax.axis_index("c")
        pltpu.sync_copy(x_hbm.at[pl.ds(wid * n_chunks_per_w, n_chunks_per_w)],
                        idx_v)
        out_row0 = wid * b_per_w

        def chunk_body(j, carry):
            pltpu.async_copy(table_hbm.at[idx_v.at[j]], rows_v, sem).wait()

            def scale_row(r, c):
                for kk in range(_EMB // _LANES):
                    sl = pl.ds(kk * _LANES, _LANES)
                    rows_v[r, sl] = rows_v[r, sl] * _SCALE
                return c

            lax.fori_loop(0, _CHUNK, scale_row, 0, unroll=2)
            pltpu.sync_copy(
                rows_v, out_hbm.at[pl.ds(out_row0 + j * _CHUNK, _CHUNK)])
            return carry

        lax.fori_loop(0, n_chunks_per_w, chunk_body, 0)

    return k(x2d, table)


def kernel(x, table):
    n_tok = x.shape[0] * x.shape[1]
    n_rows_per_w = n_tok // _NW
    n_chunks_per_w = n_rows_per_w // _CHUNK
    x2d = x.reshape(_NW * n_chunks_per_w, _CHUNK).astype(jnp.int32)
    out = _sc_embed(x2d, table, n_rows_per_w, n_chunks_per_w)
    return out.reshape(x.shape[0], x.shape[1], _EMB)


# speedup vs baseline: 1.1622x; 1.1622x over previous
import functools

import jax
import jax.numpy as jnp
from jax import lax
from jax.experimental import pallas as pl
from jax.experimental.pallas import tpu as pltpu
from jax.experimental.pallas import tpu_sc as plsc

_EMB = 64
_SCALE = _EMB ** (-0.5)
_NW = 32
_CHUNK = 128
_LANES = 16
_NBUF = 4


def _sc_embed(x2d, table, n_rows_per_w, n_chunks_per_w):
    b_per_w = n_rows_per_w
    mesh = plsc.VectorSubcoreMesh(core_axis_name="c", subcore_axis_name="s")
    total = _NW * b_per_w
    n_main = n_chunks_per_w - _NBUF

    @functools.partial(
        pl.kernel,
        mesh=mesh,
        compiler_params=pltpu.CompilerParams(use_tc_tiling_on_sc=False),
        out_type=jax.ShapeDtypeStruct((total, _EMB), jnp.float32),
        scratch_types=[
            pltpu.VMEM((n_chunks_per_w, _CHUNK), jnp.int32),
            pltpu.VMEM((_NBUF, _CHUNK, _EMB), jnp.float32),
        ]
        + [pltpu.SemaphoreType.DMA] * (2 * _NBUF),
    )
    def k(x_hbm, table_hbm, out_hbm, idx_v, rows_v, *sems):
        g_sem = sems[:_NBUF]
        o_sem = sems[_NBUF:]
        wid = lax.axis_index("s") * 2 + lax.axis_index("c")
        pltpu.sync_copy(x_hbm.at[pl.ds(wid * n_chunks_per_w, n_chunks_per_w)],
                        idx_v)
        out_row0 = wid * b_per_w

        def start_gather(j, b):
            pltpu.async_copy(table_hbm.at[idx_v.at[j]], rows_v.at[b],
                             g_sem[b])

        def wait_gather(b):
            pltpu.make_async_copy(table_hbm.at[pl.ds(0, _CHUNK)],
                                  rows_v.at[b], g_sem[b]).wait()

        def scale(b):
            def scale_row(r, c):
                for kk in range(_EMB // _LANES):
                    sl = pl.ds(kk * _LANES, _LANES)
                    rows_v[b, r, sl] = rows_v[b, r, sl] * _SCALE
                return c

            lax.fori_loop(0, _CHUNK, scale_row, 0, unroll=4)

        def start_out(j, b):
            pltpu.async_copy(
                rows_v.at[b],
                out_hbm.at[pl.ds(out_row0 + j * _CHUNK, _CHUNK)],
                o_sem[b])

        def wait_out(b):
            pltpu.make_async_copy(rows_v.at[b],
                                  out_hbm.at[pl.ds(0, _CHUNK)],
                                  o_sem[b]).wait()

        for b in range(_NBUF):
            start_gather(b, b)

        def main_body(g, carry):
            j0 = g * _NBUF
            for b in range(_NBUF):
                j = j0 + b
                wait_gather(b)
                scale(b)
                start_out(j, b)
                wait_out(b)
                start_gather(j + _NBUF, b)
            return carry

        lax.fori_loop(0, n_main // _NBUF, main_body, 0)

        for b in range(_NBUF):
            j = n_main + b
            wait_gather(b)
            scale(b)
            start_out(j, b)
        for b in range(_NBUF):
            wait_out(b)

    return k(x2d, table)


def kernel(x, table):
    n_tok = x.shape[0] * x.shape[1]
    n_rows_per_w = n_tok // _NW
    n_chunks_per_w = n_rows_per_w // _CHUNK
    x2d = x.reshape(_NW * n_chunks_per_w, _CHUNK).astype(jnp.int32)
    out = _sc_embed(x2d, table, n_rows_per_w, n_chunks_per_w)
    return out.reshape(x.shape[0], x.shape[1], _EMB)
